# Initial kernel scaffold; baseline (speedup 1.0000x reference)
import functools

import jax
import jax.numpy as jnp
from jax import lax
from jax.experimental import pallas as pl
from jax.experimental.pallas import tpu as pltpu
from jax.experimental.pallas import tpu_sc as plsc

B = 16384
O = 200
N = B * O
NC, NS = 2, 16
NW = NC * NS
E_W = N // NW
CHUNK = 3200
NCH = E_W // CHUNK
GROUPS = CHUNK // 16

INV = float(1.0 / 128.0)


def _compute_chunk(ibuf, obuf, i6, i11):
    def grp(g, carry):
        gi = g * (16 * 6)
        go = g * (16 * 11)
        xs = [plsc.load_gather(ibuf, [gi + i6 + j]) for j in range(6)]
        p = xs[4]
        c = xs[5].astype(jnp.int32)
        cg = c // 3
        cr = c % 3
        zero = jnp.zeros((16,), jnp.float32)
        for j in range(4):
            plsc.store_scatter(obuf, [go + i11 + j], xs[j] * INV)
        for j in range(3):
            plsc.store_scatter(obuf, [go + i11 + (4 + j)], jnp.where(cg == j, p, zero))
        for j in range(3):
            plsc.store_scatter(obuf, [go + i11 + (7 + j)], jnp.where(cr == j, p, zero))
        plsc.store_scatter(obuf, [go + i11 + 10], p)
        return carry

    lax.fori_loop(0, GROUPS, grp, 0)


def _body(x_hbm, out_hbm, ibuf, obuf):
    wid = lax.axis_index("s") * NC + lax.axis_index("c")
    ebase = wid * E_W
    iota = lax.iota(jnp.int32, 16)
    i6 = iota * 6
    i11 = iota * 11

    def chunk(ci, carry):
        off = ebase + ci * CHUNK
        pltpu.sync_copy(x_hbm.at[pl.ds(off * 6, CHUNK * 6)], ibuf)
        _compute_chunk(ibuf, obuf, i6, i11)
        pltpu.sync_copy(obuf, out_hbm.at[pl.ds(off * 11, CHUNK * 11)])
        return carry

    lax.fori_loop(0, NCH, chunk, 0)


@jax.jit
def kernel(x):
    xf = x.reshape(-1)
    mesh = plsc.VectorSubcoreMesh(
        core_axis_name="c", subcore_axis_name="s", num_cores=NC, num_subcores=NS
    )
    out = pl.kernel(
        _body,
        out_type=jax.ShapeDtypeStruct((N * 11,), jnp.float32),
        mesh=mesh,
        scratch_types=[
            pltpu.VMEM((CHUNK * 6,), jnp.float32),
            pltpu.VMEM((CHUNK * 11,), jnp.float32),
        ],
    )(xf)
    return out.reshape(B, O, 11)


# speedup vs baseline: 5.4474x; 5.4474x over previous
import functools

import jax
import jax.numpy as jnp
from jax import lax
from jax.experimental import pallas as pl
from jax.experimental.pallas import tpu as pltpu
from jax.experimental.pallas import tpu_sc as plsc

B = 16384
O = 200
N = B * O
NC, NS = 2, 16
NW = NC * NS
E_W = N // NW
CHUNK = 3200
NCH = E_W // CHUNK
GROUPS = CHUNK // 16

INV = float(1.0 / 128.0)


def _compute_chunk(ibuf, obuf, i6, i11):
    def grp(g, carry):
        gi = g * (16 * 6)
        go = g * (16 * 11)
        xs = [plsc.load_gather(ibuf, [gi + i6 + j]) for j in range(6)]
        p = xs[4]
        c = xs[5].astype(jnp.int32)
        cg = c // 3
        cr = c % 3
        zero = jnp.zeros((16,), jnp.float32)
        for j in range(4):
            plsc.store_scatter(obuf, [go + i11 + j], xs[j] * INV)
        for j in range(3):
            plsc.store_scatter(obuf, [go + i11 + (4 + j)], jnp.where(cg == j, p, zero))
        for j in range(3):
            plsc.store_scatter(obuf, [go + i11 + (7 + j)], jnp.where(cr == j, p, zero))
        plsc.store_scatter(obuf, [go + i11 + 10], p)
        return carry

    lax.fori_loop(0, GROUPS, grp, 0)


def _body(x_hbm, out_hbm, ibuf, obuf):
    wid = lax.axis_index("s") * NC + lax.axis_index("c")
    ebase = wid * E_W
    iota = lax.iota(jnp.int32, 16)
    i6 = iota * 6
    i11 = iota * 11

    def chunk(ci, carry):
        off = ebase + ci * CHUNK
        pltpu.sync_copy(x_hbm.at[pl.ds(off * 6, CHUNK * 6)], ibuf)
        _compute_chunk(ibuf, obuf, i6, i11)
        pltpu.sync_copy(obuf, out_hbm.at[pl.ds(off * 11, CHUNK * 11)])
        return carry

    lax.fori_loop(0, NCH, chunk, 0)


@jax.jit
def kernel(x):
    xf = x.reshape(-1)
    mesh = plsc.VectorSubcoreMesh(
        core_axis_name="c", subcore_axis_name="s", num_cores=NC, num_subcores=NS
    )
    out = pl.kernel(
        _body,
        out_type=jax.ShapeDtypeStruct((N * 11,), jnp.float32),
        mesh=mesh,
        scratch_types=[
            pltpu.VMEM((CHUNK * 6,), jnp.float32),
            pltpu.VMEM((CHUNK * 11,), jnp.float32),
        ],
        compiler_params=pltpu.CompilerParams(needs_layout_passes=False),
    )(xf)
    return out.reshape(B, O, 11)


# speedup vs baseline: 5.6278x; 1.0331x over previous
import functools

import jax
import jax.numpy as jnp
from jax import lax
from jax.experimental import pallas as pl
from jax.experimental.pallas import tpu as pltpu
from jax.experimental.pallas import tpu_sc as plsc

B = 16384
O = 200
N = B * O
NC, NS = 2, 16
NW = NC * NS
E_W = N // NW
CHUNK = 3200
NCH = E_W // CHUNK
GROUPS = CHUNK // 16

INV = float(1.0 / 128.0)


def _compute_chunk(ibuf, obuf, i6, i11):
    def grp(g):
        gi = g * (16 * 6)
        go = g * (16 * 11)
        xs = [plsc.load_gather(ibuf, [gi + i6 + j]) for j in range(6)]
        p = xs[4]
        c = xs[5].astype(jnp.int32)
        cg = c // 3
        cr = c % 3
        zero = jnp.zeros((16,), jnp.float32)
        for j in range(4):
            plsc.store_scatter(obuf, [go + i11 + j], xs[j] * INV)
        for j in range(3):
            plsc.store_scatter(obuf, [go + i11 + (4 + j)], jnp.where(cg == j, p, zero))
        for j in range(3):
            plsc.store_scatter(obuf, [go + i11 + (7 + j)], jnp.where(cr == j, p, zero))
        plsc.store_scatter(obuf, [go + i11 + 10], p)

    plsc.parallel_loop(0, GROUPS, 1, unroll=8)(grp)


def _body(x_hbm, out_hbm, ibuf, obuf):
    wid = lax.axis_index("s") * NC + lax.axis_index("c")
    ebase = wid * E_W
    iota = lax.iota(jnp.int32, 16)
    i6 = iota * 6
    i11 = iota * 11

    def chunk(ci, carry):
        off = ebase + ci * CHUNK
        pltpu.sync_copy(x_hbm.at[pl.ds(off * 6, CHUNK * 6)], ibuf)
        _compute_chunk(ibuf, obuf, i6, i11)
        pltpu.sync_copy(obuf, out_hbm.at[pl.ds(off * 11, CHUNK * 11)])
        return carry

    lax.fori_loop(0, NCH, chunk, 0)


@jax.jit
def kernel(x):
    xf = x.reshape(-1)
    mesh = plsc.VectorSubcoreMesh(
        core_axis_name="c", subcore_axis_name="s", num_cores=NC, num_subcores=NS
    )
    out = pl.kernel(
        _body,
        out_type=jax.ShapeDtypeStruct((N * 11,), jnp.float32),
        mesh=mesh,
        scratch_types=[
            pltpu.VMEM((CHUNK * 6,), jnp.float32),
            pltpu.VMEM((CHUNK * 11,), jnp.float32),
        ],
        compiler_params=pltpu.CompilerParams(needs_layout_passes=False),
    )(xf)
    return out.reshape(B, O, 11)


# speedup vs baseline: 65.1392x; 11.5745x over previous
import functools

import jax
import jax.numpy as jnp
from jax import lax
from jax.experimental import pallas as pl
from jax.experimental.pallas import tpu as pltpu
from jax.experimental.pallas import tpu_sc as plsc

B = 16384
O = 200
P = B * O
NC, NS = 2, 16
NW = NC * NS
E_W = P // NW
CHUNK = 3200
NCH = E_W // CHUNK
GROUPS = CHUNK // 16

INV = float(1.0 / 128.0)


def _compute_chunk(ibuf, obuf):
    def grp(g):
        t = g * 16
        xs = [ibuf[pl.ds(j * CHUNK + t, 16)] for j in range(6)]
        p = xs[4]
        c = xs[5].astype(jnp.int32)
        cg = c // 3
        cr = c % 3
        zero = jnp.zeros((16,), jnp.float32)
        for j in range(4):
            obuf[pl.ds(j * CHUNK + t, 16)] = xs[j] * INV
        for j in range(3):
            obuf[pl.ds((4 + j) * CHUNK + t, 16)] = jnp.where(cg == j, p, zero)
        for j in range(3):
            obuf[pl.ds((7 + j) * CHUNK + t, 16)] = jnp.where(cr == j, p, zero)
        obuf[pl.ds(10 * CHUNK + t, 16)] = p

    plsc.parallel_loop(0, GROUPS, 1, unroll=8)(grp)


def _body(x_hbm, out_hbm, ibuf, obuf):
    wid = lax.axis_index("s") * NC + lax.axis_index("c")
    wbase = wid * E_W

    def chunk(ci, carry):
        off = wbase + ci * CHUNK
        for j in range(6):
            pltpu.sync_copy(
                x_hbm.at[pl.ds(j * P + off, CHUNK)], ibuf.at[pl.ds(j * CHUNK, CHUNK)]
            )
        _compute_chunk(ibuf, obuf)
        for j in range(11):
            pltpu.sync_copy(
                obuf.at[pl.ds(j * CHUNK, CHUNK)], out_hbm.at[pl.ds(j * P + off, CHUNK)]
            )
        return carry

    lax.fori_loop(0, NCH, chunk, 0)


@jax.jit
def kernel(x):
    xf = (
        x.reshape(B // 128, 128, O // 8, 8, 6)
        .transpose(4, 2, 0, 3, 1)
        .reshape(-1)
    )
    mesh = plsc.VectorSubcoreMesh(
        core_axis_name="c", subcore_axis_name="s", num_cores=NC, num_subcores=NS
    )
    outf = pl.kernel(
        _body,
        out_type=jax.ShapeDtypeStruct((11 * P,), jnp.float32),
        mesh=mesh,
        scratch_types=[
            pltpu.VMEM((6 * CHUNK,), jnp.float32),
            pltpu.VMEM((11 * CHUNK,), jnp.float32),
        ],
        compiler_params=pltpu.CompilerParams(needs_layout_passes=False),
    )(xf)
    return (
        outf.reshape(11, O // 8, B // 128, 8, 128)
        .transpose(2, 4, 1, 3, 0)
        .reshape(B, O, 11)
    )
